# baseline (speedup 1.0000x reference)
import functools

import jax
import jax.numpy as jnp
from jax import lax
from jax.experimental import pallas as pl
from jax.experimental.pallas import tpu as pltpu
from jax.experimental.pallas import tpu_sc as plsc

_NC = 2
_NS = 16
_NW = _NC * _NS

_CH = 128


def _cdiv(a, b):
    return (a + b - 1) // b


def _make_src(central_species, n_pad_rows, cols, b1, safe):
    n = central_species.shape[0]
    pad = n_pad_rows * cols - n
    sp = jnp.concatenate(
        [central_species, jnp.ones((pad,), dtype=central_species.dtype)]
    ).reshape(n_pad_rows, cols)

    def body(sp_ref, src_ref):
        z = (sp_ref[...] == 0).astype(jnp.float32)
        r, c = z.shape
        tri = (
            lax.broadcasted_iota(jnp.int32, (c, c), 0)
            <= lax.broadcasted_iota(jnp.int32, (c, c), 1)
        ).astype(jnp.float32)
        y = jnp.dot(z, tri, preferred_element_type=jnp.float32)
        s = jnp.sum(z, axis=1, keepdims=True)
        low = (
            lax.broadcasted_iota(jnp.int32, (r, r), 1)
            < lax.broadcasted_iota(jnp.int32, (r, r), 0)
        ).astype(jnp.float32)
        off = jnp.dot(low, s, preferred_element_type=jnp.float32)
        cz = y + off - z
        gi = (
            lax.broadcasted_iota(jnp.int32, (r, c), 0) * c
            + lax.broadcasted_iota(jnp.int32, (r, c), 1)
        ).astype(jnp.float32)
        srcf = jnp.where(z > 0.5, cz, b1 + gi - cz)
        srcf = jnp.where(gi < n, srcf, safe + gi - n)
        src_ref[...] = srcf.astype(jnp.int32)

    return pl.pallas_call(
        body,
        out_shape=jax.ShapeDtypeStruct((n_pad_rows, cols), jnp.int32),
    )(sp)


def _make_inv(src2d, ni):
    n_rows = src2d.shape[0]
    assert n_rows % _NS == 0
    rps = n_rows // _NS
    assert ni % (2 * _NS * 8) == 0
    half = ni // 2
    opc = half // _NS
    mesh = plsc.VectorSubcoreMesh(core_axis_name="c", subcore_axis_name="s")

    @functools.partial(
        pl.kernel,
        mesh=mesh,
        out_type=jax.ShapeDtypeStruct((ni,), jnp.int32),
        scratch_types=[
            pltpu.VMEM((rps, _CH), jnp.int32),
            pltpu.VMEM((rps, _CH), jnp.int32),
            pltpu.VMEM((opc,), jnp.int32),
            pltpu.VMEM_SHARED((ni,), jnp.int32),
            pltpu.SemaphoreType.DMA,
        ],
    )
    def invert(src_hbm, inv_hbm, srcv, posv, stage, shared, sem):
        cid = lax.axis_index("c")
        sid = lax.axis_index("s")
        r0 = sid * rps
        pltpu.sync_copy(src_hbm.at[pl.ds(r0, rps)], srcv)
        lane = lax.iota(jnp.int32, 16)
        a0 = r0 * _CH

        def build(j, carry):
            for m in range(_CH // 16):
                posv[j, pl.ds(16 * m, 16)] = a0 + j * _CH + 16 * m + lane
            return carry

        lax.fori_loop(0, rps, build, 0)

        def fire(j, carry):
            pltpu.async_copy(posv.at[j], shared.at[srcv.at[j]], sem)
            return carry

        lax.fori_loop(0, rps, fire, 0)

        def drain(j, carry):
            pltpu.make_async_copy(
                posv.at[0], shared.at[srcv.at[0]], sem
            ).wait()
            return carry

        lax.fori_loop(0, rps, drain, 0)
        plsc.subcore_barrier()

        g = cid * half + sid * opc
        pltpu.sync_copy(shared.at[pl.ds(g, opc)], stage)
        pltpu.sync_copy(stage, inv_hbm.at[pl.ds(g, opc)])

    return invert(src2d)


def _scatter_rows(features_0, features_1, inv_flat, b1):
    n0, d = features_0.shape
    n1 = features_1.shape[0]
    n = n0 + n1
    nf0 = n0 // _CH
    t0 = n0 - nf0 * _CH
    nf1 = n1 // _CH
    t1 = n1 - nf1 * _CH
    trips0 = _cdiv(nf0, _NW)
    trips1 = _cdiv(nf1, _NW)
    st0 = max(t0, 16)
    st1 = max(t1, 16)
    mesh = plsc.VectorSubcoreMesh(core_axis_name="c", subcore_axis_name="s")

    @functools.partial(
        pl.kernel,
        mesh=mesh,
        out_type=jax.ShapeDtypeStruct((n, d), jnp.float32),
        scratch_types=[
            pltpu.VMEM((_CH,), jnp.int32),
            pltpu.VMEM((_CH,), jnp.int32),
            pltpu.VMEM((_CH,), jnp.int32),
            pltpu.VMEM((_CH, d), jnp.float32),
            pltpu.VMEM((_CH, d), jnp.float32),
            pltpu.VMEM((_CH, d), jnp.float32),
            pltpu.VMEM((st0,), jnp.int32),
            pltpu.VMEM((st0, d), jnp.float32),
            pltpu.VMEM((st1,), jnp.int32),
            pltpu.VMEM((st1, d), jnp.float32),
            pltpu.SemaphoreType.DMA,
            pltpu.SemaphoreType.DMA,
            pltpu.SemaphoreType.DMA,
            pltpu.SemaphoreType.DMA,
            pltpu.SemaphoreType.DMA,
            pltpu.SemaphoreType.DMA,
            pltpu.SemaphoreType.DMA,
        ],
    )
    def scatter(
        f0_hbm, f1_hbm, invf_hbm, out_hbm,
        invv0, invv1, invv2, rows0, rows1, rows2,
        invv_t0, rows_t0, invv_t1, rows_t1,
        rsem0, rsem1, rsem2, ssem0, ssem1, ssem2, sem_t,
    ):
        wid = lax.axis_index("s") * _NC + lax.axis_index("c")
        bufs = (
            (invv0, rows0, rsem0, ssem0),
            (invv1, rows1, rsem1, ssem1),
            (invv2, rows2, rsem2, ssem2),
        )

        def table_loop(feat_hbm, rb, nf, trips):
            def step(i, b):
                invv, rows, rsem, ssem = bufs[b]
                k = wid + _NW * i
                k = jnp.where(k >= nf, k - nf, k)

                @pl.when(i >= 3)
                def _():
                    pltpu.make_async_copy(
                        rows, out_hbm.at[invv], ssem
                    ).wait()

                cpi = pltpu.async_copy(
                    invf_hbm.at[pl.ds(rb + k * _CH, _CH)], invv, rsem
                )
                cpr = pltpu.async_copy(
                    feat_hbm.at[pl.ds(k * _CH, _CH)], rows, rsem
                )
                cpi.wait()
                cpr.wait()
                pltpu.async_copy(rows, out_hbm.at[invv], ssem)

            def body(i, carry):
                for b in range(3):
                    @pl.when(i % 3 == b)
                    def _(b=b):
                        step(i, b)

                return carry

            lax.fori_loop(0, trips, body, 0)
            for b in range(3):
                invv, rows, rsem, ssem = bufs[b]
                pltpu.make_async_copy(rows, out_hbm.at[invv], ssem).wait()

        table_loop(f0_hbm, 0, nf0, trips0)
        table_loop(f1_hbm, b1, nf1, trips1)

        if t0:
            @pl.when(wid == 0)
            def _():
                r0 = nf0 * _CH
                pltpu.sync_copy(invf_hbm.at[pl.ds(r0, t0)], invv_t0)
                pltpu.sync_copy(f0_hbm.at[pl.ds(r0, t0)], rows_t0)
                pltpu.async_copy(rows_t0, out_hbm.at[invv_t0], sem_t).wait()

        if t1:
            @pl.when(wid == 1)
            def _():
                r1 = nf1 * _CH
                pltpu.sync_copy(
                    invf_hbm.at[pl.ds(b1 + r1, t1)], invv_t1
                )
                pltpu.sync_copy(f1_hbm.at[pl.ds(r1, t1)], rows_t1)
                pltpu.async_copy(rows_t1, out_hbm.at[invv_t1], sem_t).wait()

    return scatter(features_0, features_1, inv_flat)


def kernel(features_0, features_1, central_species):
    n0, d = features_0.shape
    n1 = features_1.shape[0]
    n = central_species.shape[0]
    cols = 256
    chunk_rows = _cdiv(_cdiv(n, _CH), _NW * 8) * _NW * 8
    np_total = chunk_rows * _CH
    rows = np_total // cols
    b1 = _cdiv(n0, _CH) * _CH
    safe = b1 + _cdiv(n1, _CH) * _CH
    ni = _cdiv(safe + (np_total - n), _CH) * _CH

    src = _make_src(central_species, rows, cols, b1, safe)
    src2d = src.reshape(chunk_rows, _CH)
    inv = _make_inv(src2d, ni)
    return _scatter_rows(features_0, features_1, inv, b1)


# speedup vs baseline: 1.0081x; 1.0081x over previous
import functools

import jax
import jax.numpy as jnp
from jax import lax
from jax.experimental import pallas as pl
from jax.experimental.pallas import tpu as pltpu
from jax.experimental.pallas import tpu_sc as plsc

_NC = 2
_NS = 16
_NW = _NC * _NS

_CH = 128


def _cdiv(a, b):
    return (a + b - 1) // b


def _make_src(central_species, n_pad_rows, cols, b1, safe):
    n = central_species.shape[0]
    pad = n_pad_rows * cols - n
    sp = jnp.concatenate(
        [central_species, jnp.ones((pad,), dtype=central_species.dtype)]
    ).reshape(n_pad_rows, cols)

    def body(sp_ref, src_ref):
        z = (sp_ref[...] == 0).astype(jnp.float32)
        r, c = z.shape
        tri = (
            lax.broadcasted_iota(jnp.int32, (c, c), 0)
            <= lax.broadcasted_iota(jnp.int32, (c, c), 1)
        ).astype(jnp.float32)
        y = jnp.dot(z, tri, preferred_element_type=jnp.float32)
        s = jnp.sum(z, axis=1, keepdims=True)
        low = (
            lax.broadcasted_iota(jnp.int32, (r, r), 1)
            < lax.broadcasted_iota(jnp.int32, (r, r), 0)
        ).astype(jnp.float32)
        off = jnp.dot(low, s, preferred_element_type=jnp.float32)
        cz = y + off - z
        gi = (
            lax.broadcasted_iota(jnp.int32, (r, c), 0) * c
            + lax.broadcasted_iota(jnp.int32, (r, c), 1)
        ).astype(jnp.float32)
        srcf = jnp.where(z > 0.5, cz, b1 + gi - cz)
        srcf = jnp.where(gi < n, srcf, safe + gi - n)
        src_ref[...] = srcf.astype(jnp.int32)

    return pl.pallas_call(
        body,
        out_shape=jax.ShapeDtypeStruct((n_pad_rows, cols), jnp.int32),
    )(sp)


def _make_inv(src2d, ni):
    n_rows = src2d.shape[0]
    assert n_rows % _NS == 0
    rps = n_rows // _NS
    assert ni % (2 * _NS * 8) == 0
    half = ni // 2
    opc = half // _NS
    mesh = plsc.VectorSubcoreMesh(core_axis_name="c", subcore_axis_name="s")

    @functools.partial(
        pl.kernel,
        mesh=mesh,
        out_type=jax.ShapeDtypeStruct((ni,), jnp.int32),
        scratch_types=[
            pltpu.VMEM((rps, _CH), jnp.int32),
            pltpu.VMEM((rps, _CH), jnp.int32),
            pltpu.VMEM((opc,), jnp.int32),
            pltpu.VMEM_SHARED((ni,), jnp.int32),
            pltpu.SemaphoreType.DMA,
        ],
    )
    def invert(src_hbm, inv_hbm, srcv, posv, stage, shared, sem):
        cid = lax.axis_index("c")
        sid = lax.axis_index("s")
        r0 = sid * rps
        pltpu.sync_copy(src_hbm.at[pl.ds(r0, rps)], srcv)
        lane = lax.iota(jnp.int32, 16)
        a0 = r0 * _CH

        def build(j, carry):
            for m in range(_CH // 16):
                posv[j, pl.ds(16 * m, 16)] = a0 + j * _CH + 16 * m + lane
            return carry

        lax.fori_loop(0, rps, build, 0)

        def fire(j, carry):
            pltpu.async_copy(posv.at[j], shared.at[srcv.at[j]], sem)
            return carry

        lax.fori_loop(0, rps, fire, 0)

        def drain(j, carry):
            pltpu.make_async_copy(
                posv.at[0], shared.at[srcv.at[0]], sem
            ).wait()
            return carry

        lax.fori_loop(0, rps, drain, 0)
        plsc.subcore_barrier()

        g = cid * half + sid * opc
        pltpu.sync_copy(shared.at[pl.ds(g, opc)], stage)
        pltpu.sync_copy(stage, inv_hbm.at[pl.ds(g, opc)])

    return invert(src2d)


def _scatter_rows(features_0, features_1, inv_flat, b1):
    n0, d = features_0.shape
    n1 = features_1.shape[0]
    n = n0 + n1
    nf0 = n0 // _CH
    t0 = n0 - nf0 * _CH
    nf1 = n1 // _CH
    t1 = n1 - nf1 * _CH
    trips0 = _cdiv(nf0, _NW)
    trips1 = _cdiv(nf1, _NW)
    st0 = max(t0, 16)
    st1 = max(t1, 16)
    mesh = plsc.VectorSubcoreMesh(core_axis_name="c", subcore_axis_name="s")

    @functools.partial(
        pl.kernel,
        mesh=mesh,
        out_type=jax.ShapeDtypeStruct((n, d), jnp.float32),
        scratch_types=[
            pltpu.VMEM((_CH,), jnp.int32),
            pltpu.VMEM((_CH,), jnp.int32),
            pltpu.VMEM((_CH,), jnp.int32),
            pltpu.VMEM((_CH, d), jnp.float32),
            pltpu.VMEM((_CH, d), jnp.float32),
            pltpu.VMEM((_CH, d), jnp.float32),
            pltpu.VMEM((st0,), jnp.int32),
            pltpu.VMEM((st0, d), jnp.float32),
            pltpu.VMEM((st1,), jnp.int32),
            pltpu.VMEM((st1, d), jnp.float32),
            pltpu.SemaphoreType.DMA,
            pltpu.SemaphoreType.DMA,
            pltpu.SemaphoreType.DMA,
            pltpu.SemaphoreType.DMA,
            pltpu.SemaphoreType.DMA,
            pltpu.SemaphoreType.DMA,
            pltpu.SemaphoreType.DMA,
        ],
    )
    def scatter(
        f0_hbm, f1_hbm, invf_hbm, out_hbm,
        invv0, invv1, invv2, rows0, rows1, rows2,
        invv_t0, rows_t0, invv_t1, rows_t1,
        rsem0, rsem1, rsem2, ssem0, ssem1, ssem2, sem_t,
    ):
        wid = lax.axis_index("s") * _NC + lax.axis_index("c")
        bufs = (
            (invv0, rows0, rsem0, ssem0),
            (invv1, rows1, rsem1, ssem1),
            (invv2, rows2, rsem2, ssem2),
        )

        def table_loop(feat_hbm, rb, nf, trips):
            base_t = nf // _NW
            rem_t = nf - base_t * _NW
            del trips
            def step(i, b):
                invv, rows, rsem, ssem = bufs[b]
                k = wid + _NW * i
                k = jnp.where(k >= nf, k - nf, k)

                @pl.when(i >= 3)
                def _():
                    pltpu.make_async_copy(
                        rows, out_hbm.at[invv], ssem
                    ).wait()

                cpi = pltpu.async_copy(
                    invf_hbm.at[pl.ds(rb + k * _CH, _CH)], invv, rsem
                )
                cpr = pltpu.async_copy(
                    feat_hbm.at[pl.ds(k * _CH, _CH)], rows, rsem
                )
                cpi.wait()
                cpr.wait()
                pltpu.async_copy(rows, out_hbm.at[invv], ssem)

            def body(i, carry):
                for b in range(3):
                    @pl.when(i % 3 == b)
                    def _(b=b):
                        step(i, b)

                return carry

            my_trips = base_t + jnp.where(wid < rem_t, 1, 0)
            lax.fori_loop(0, my_trips, body, 0)
            for b in range(3):
                invv, rows, rsem, ssem = bufs[b]
                pltpu.make_async_copy(rows, out_hbm.at[invv], ssem).wait()

        table_loop(f0_hbm, 0, nf0, trips0)
        table_loop(f1_hbm, b1, nf1, trips1)

        if t0:
            @pl.when(wid == 0)
            def _():
                r0 = nf0 * _CH
                pltpu.sync_copy(invf_hbm.at[pl.ds(r0, t0)], invv_t0)
                pltpu.sync_copy(f0_hbm.at[pl.ds(r0, t0)], rows_t0)
                pltpu.async_copy(rows_t0, out_hbm.at[invv_t0], sem_t).wait()

        if t1:
            @pl.when(wid == 1)
            def _():
                r1 = nf1 * _CH
                pltpu.sync_copy(
                    invf_hbm.at[pl.ds(b1 + r1, t1)], invv_t1
                )
                pltpu.sync_copy(f1_hbm.at[pl.ds(r1, t1)], rows_t1)
                pltpu.async_copy(rows_t1, out_hbm.at[invv_t1], sem_t).wait()

    return scatter(features_0, features_1, inv_flat)


def kernel(features_0, features_1, central_species):
    n0, d = features_0.shape
    n1 = features_1.shape[0]
    n = central_species.shape[0]
    cols = 256
    chunk_rows = _cdiv(_cdiv(n, _CH), _NW * 8) * _NW * 8
    np_total = chunk_rows * _CH
    rows = np_total // cols
    b1 = _cdiv(n0, _CH) * _CH
    safe = b1 + _cdiv(n1, _CH) * _CH
    ni = _cdiv(safe + (np_total - n), _CH) * _CH

    src = _make_src(central_species, rows, cols, b1, safe)
    src2d = src.reshape(chunk_rows, _CH)
    inv = _make_inv(src2d, ni)
    return _scatter_rows(features_0, features_1, inv, b1)
